# Initial kernel scaffold; baseline (speedup 1.0000x reference)
import jax
import jax.numpy as jnp
from jax.experimental import pallas as pl


def kernel(input, Nu_uv, Nv_uv, uspan_uv, vspan_uv):
    raise NotImplementedError("write your pallas kernel here")


# speedup vs baseline: 233.4022x; 233.4022x over previous
import jax
import jax.numpy as jnp
from jax.experimental import pallas as pl

_P = 3
_Q = 3


def _surf_kernel(nut_ref, nvt_ref, iu_ref, iv_ref, x_ref, out_ref):
    M = x_ref.shape[2]
    N = x_ref.shape[3]
    OUT = out_ref.shape[2]

    m_idx = jax.lax.broadcasted_iota(jnp.int32, (M, OUT), 0)
    iu = iu_ref[0, :]
    bu = jnp.zeros((M, OUT), jnp.float32)
    for l in range(_P + 1):
        bu = bu + jnp.where(m_idx == (iu[None, :] + l), nut_ref[l, :][None, :], 0.0)

    n_idx = jax.lax.broadcasted_iota(jnp.int32, (N, OUT), 0)
    iv = iv_ref[0, :]
    bv = jnp.zeros((N, OUT), jnp.float32)
    for r in range(_Q + 1):
        bv = bv + jnp.where(n_idx == (iv[None, :] + r), nvt_ref[r, :][None, :], 0.0)

    s = []
    for d in range(4):
        xd = x_ref[0, d]
        tmp = jax.lax.dot_general(
            bu, xd, (((0,), (0,)), ((), ())),
            precision=jax.lax.Precision.HIGHEST,
            preferred_element_type=jnp.float32)
        sd = jax.lax.dot_general(
            tmp, bv, (((1,), (0,)), ((), ())),
            precision=jax.lax.Precision.HIGHEST,
            preferred_element_type=jnp.float32)
        s.append(sd)
    w = s[3]
    for d in range(3):
        out_ref[0, d] = s[d] / w


def kernel(input, Nu_uv, Nv_uv, uspan_uv, vspan_uv):
    Bsz, M, N, _ = input.shape
    OUT = uspan_uv.shape[0]

    nut = Nu_uv[:, 0, :].T.astype(jnp.float32)
    nvt = Nv_uv[0, :, :].T.astype(jnp.float32)
    iu = (uspan_uv[:, 0] - _P).astype(jnp.int32).reshape(1, OUT)
    iv = (vspan_uv[0, :] - _Q).astype(jnp.int32).reshape(1, OUT)
    xp = jnp.transpose(input, (0, 3, 1, 2))

    out = pl.pallas_call(
        _surf_kernel,
        grid=(Bsz,),
        in_specs=[
            pl.BlockSpec((_P + 1, OUT), lambda b: (0, 0)),
            pl.BlockSpec((_Q + 1, OUT), lambda b: (0, 0)),
            pl.BlockSpec((1, OUT), lambda b: (0, 0)),
            pl.BlockSpec((1, OUT), lambda b: (0, 0)),
            pl.BlockSpec((1, 4, M, N), lambda b: (b, 0, 0, 0)),
        ],
        out_specs=pl.BlockSpec((1, 3, OUT, OUT), lambda b: (b, 0, 0, 0)),
        out_shape=jax.ShapeDtypeStruct((Bsz, 3, OUT, OUT), jnp.float32),
    )(nut, nvt, iu, iv, xp)
    return jnp.transpose(out, (0, 2, 3, 1))


# speedup vs baseline: 351.9963x; 1.5081x over previous
import jax
import jax.numpy as jnp
from jax.experimental import pallas as pl

_P = 3
_Q = 3


def _surf_kernel(nut_ref, nvt_ref, iu_ref, iv_ref, x_ref, out_ref):
    M = x_ref.shape[2]
    N = x_ref.shape[3]
    OUT = out_ref.shape[2]

    m_idx = jax.lax.broadcasted_iota(jnp.int32, (M, OUT), 0)
    iu = iu_ref[0, :]
    bu = jnp.zeros((M, OUT), jnp.float32)
    for l in range(_P + 1):
        bu = bu + jnp.where(m_idx == (iu[None, :] + l), nut_ref[l, :][None, :], 0.0)

    n_idx = jax.lax.broadcasted_iota(jnp.int32, (N, OUT), 0)
    iv = iv_ref[0, :]
    bv = jnp.zeros((N, OUT), jnp.float32)
    for r in range(_Q + 1):
        bv = bv + jnp.where(n_idx == (iv[None, :] + r), nvt_ref[r, :][None, :], 0.0)

    s = []
    for d in range(4):
        xd = x_ref[0, d]
        tmp = jax.lax.dot_general(
            bu, xd, (((0,), (0,)), ((), ())),
            precision=jax.lax.Precision.DEFAULT,
            preferred_element_type=jnp.float32)
        sd = jax.lax.dot_general(
            tmp, bv, (((1,), (0,)), ((), ())),
            precision=jax.lax.Precision.DEFAULT,
            preferred_element_type=jnp.float32)
        s.append(sd)
    w = s[3]
    for d in range(3):
        out_ref[0, d] = s[d] / w


def kernel(input, Nu_uv, Nv_uv, uspan_uv, vspan_uv):
    Bsz, M, N, _ = input.shape
    OUT = uspan_uv.shape[0]

    nut = Nu_uv[:, 0, :].T.astype(jnp.float32)
    nvt = Nv_uv[0, :, :].T.astype(jnp.float32)
    iu = (uspan_uv[:, 0] - _P).astype(jnp.int32).reshape(1, OUT)
    iv = (vspan_uv[0, :] - _Q).astype(jnp.int32).reshape(1, OUT)
    xp = jnp.transpose(input, (0, 3, 1, 2))

    out = pl.pallas_call(
        _surf_kernel,
        grid=(Bsz,),
        in_specs=[
            pl.BlockSpec((_P + 1, OUT), lambda b: (0, 0)),
            pl.BlockSpec((_Q + 1, OUT), lambda b: (0, 0)),
            pl.BlockSpec((1, OUT), lambda b: (0, 0)),
            pl.BlockSpec((1, OUT), lambda b: (0, 0)),
            pl.BlockSpec((1, 4, M, N), lambda b: (b, 0, 0, 0)),
        ],
        out_specs=pl.BlockSpec((1, 3, OUT, OUT), lambda b: (b, 0, 0, 0)),
        out_shape=jax.ShapeDtypeStruct((Bsz, 3, OUT, OUT), jnp.float32),
    )(nut, nvt, iu, iv, xp)
    return jnp.transpose(out, (0, 2, 3, 1))


# speedup vs baseline: 442.4346x; 1.2569x over previous
import jax
import jax.numpy as jnp
from jax.experimental import pallas as pl

_P = 3
_Q = 3


def _surf_kernel(nut_ref, nvt_ref, iu_ref, iv_ref, x_ref, out_ref):
    M = x_ref.shape[2]
    N = x_ref.shape[3]
    OUT = out_ref.shape[2]

    m_idx = jax.lax.broadcasted_iota(jnp.int32, (M, OUT), 0)
    iu = iu_ref[0, :]
    bu = jnp.zeros((M, OUT), jnp.float32)
    for l in range(_P + 1):
        bu = bu + jnp.where(m_idx == (iu[None, :] + l), nut_ref[l, :][None, :], 0.0)

    n_idx = jax.lax.broadcasted_iota(jnp.int32, (N, OUT), 0)
    iv = iv_ref[0, :]
    bv = jnp.zeros((N, OUT), jnp.float32)
    for r in range(_Q + 1):
        bv = bv + jnp.where(n_idx == (iv[None, :] + r), nvt_ref[r, :][None, :], 0.0)

    for b in range(x_ref.shape[0]):
        s = []
        for d in range(4):
            xd = x_ref[b, d]
            tmp = jax.lax.dot_general(
                bu, xd, (((0,), (0,)), ((), ())),
                precision=jax.lax.Precision.DEFAULT,
                preferred_element_type=jnp.float32)
            sd = jax.lax.dot_general(
                tmp, bv, (((1,), (0,)), ((), ())),
                precision=jax.lax.Precision.DEFAULT,
                preferred_element_type=jnp.float32)
            s.append(sd)
        w = s[3]
        for d in range(3):
            out_ref[b, d] = s[d] / w


def kernel(input, Nu_uv, Nv_uv, uspan_uv, vspan_uv):
    Bsz, M, N, _ = input.shape
    OUT = uspan_uv.shape[0]

    nut = Nu_uv[:, 0, :].T.astype(jnp.float32)
    nvt = Nv_uv[0, :, :].T.astype(jnp.float32)
    iu = (uspan_uv[:, 0] - _P).astype(jnp.int32).reshape(1, OUT)
    iv = (vspan_uv[0, :] - _Q).astype(jnp.int32).reshape(1, OUT)
    xp = jnp.transpose(input, (0, 3, 1, 2))

    BT = 4
    out = pl.pallas_call(
        _surf_kernel,
        grid=(Bsz // BT,),
        in_specs=[
            pl.BlockSpec((_P + 1, OUT), lambda b: (0, 0)),
            pl.BlockSpec((_Q + 1, OUT), lambda b: (0, 0)),
            pl.BlockSpec((1, OUT), lambda b: (0, 0)),
            pl.BlockSpec((1, OUT), lambda b: (0, 0)),
            pl.BlockSpec((BT, 4, M, N), lambda b: (b, 0, 0, 0)),
        ],
        out_specs=pl.BlockSpec((BT, 3, OUT, OUT), lambda b: (b, 0, 0, 0)),
        out_shape=jax.ShapeDtypeStruct((Bsz, 3, OUT, OUT), jnp.float32),
    )(nut, nvt, iu, iv, xp)
    return jnp.transpose(out, (0, 2, 3, 1))


# speedup vs baseline: 449.5023x; 1.0160x over previous
import jax
import jax.numpy as jnp
from jax.experimental import pallas as pl

_P = 3
_Q = 3


def _surf_kernel(nut_ref, nvt_ref, iu_ref, iv_ref, x_ref, out_ref):
    M = x_ref.shape[2]
    N = x_ref.shape[3]
    OUT = out_ref.shape[2]

    m_idx = jax.lax.broadcasted_iota(jnp.int32, (M, OUT), 0)
    iu = iu_ref[0, :]
    bu = jnp.zeros((M, OUT), jnp.float32)
    for l in range(_P + 1):
        bu = bu + jnp.where(m_idx == (iu[None, :] + l), nut_ref[l, :][None, :], 0.0)

    n_idx = jax.lax.broadcasted_iota(jnp.int32, (N, OUT), 0)
    iv = iv_ref[0, :]
    bv = jnp.zeros((N, OUT), jnp.float32)
    for r in range(_Q + 1):
        bv = bv + jnp.where(n_idx == (iv[None, :] + r), nvt_ref[r, :][None, :], 0.0)

    for b in range(x_ref.shape[0]):
        s = []
        for d in range(4):
            xd = x_ref[b, d]
            tmp = jax.lax.dot_general(
                bu, xd, (((0,), (0,)), ((), ())),
                precision=jax.lax.Precision.DEFAULT,
                preferred_element_type=jnp.float32)
            sd = jax.lax.dot_general(
                tmp, bv, (((1,), (0,)), ((), ())),
                precision=jax.lax.Precision.DEFAULT,
                preferred_element_type=jnp.float32)
            s.append(sd)
        w = s[3]
        for d in range(3):
            out_ref[b, d] = s[d] / w


def kernel(input, Nu_uv, Nv_uv, uspan_uv, vspan_uv):
    Bsz, M, N, _ = input.shape
    OUT = uspan_uv.shape[0]

    nut = Nu_uv[:, 0, :].T.astype(jnp.float32)
    nvt = Nv_uv[0, :, :].T.astype(jnp.float32)
    iu = (uspan_uv[:, 0] - _P).astype(jnp.int32).reshape(1, OUT)
    iv = (vspan_uv[0, :] - _Q).astype(jnp.int32).reshape(1, OUT)
    xp = jnp.transpose(input, (0, 3, 1, 2))

    BT = 8
    out = pl.pallas_call(
        _surf_kernel,
        grid=(Bsz // BT,),
        in_specs=[
            pl.BlockSpec((_P + 1, OUT), lambda b: (0, 0)),
            pl.BlockSpec((_Q + 1, OUT), lambda b: (0, 0)),
            pl.BlockSpec((1, OUT), lambda b: (0, 0)),
            pl.BlockSpec((1, OUT), lambda b: (0, 0)),
            pl.BlockSpec((BT, 4, M, N), lambda b: (b, 0, 0, 0)),
        ],
        out_specs=pl.BlockSpec((BT, 3, OUT, OUT), lambda b: (b, 0, 0, 0)),
        out_shape=jax.ShapeDtypeStruct((Bsz, 3, OUT, OUT), jnp.float32),
    )(nut, nvt, iu, iv, xp)
    return jnp.transpose(out, (0, 2, 3, 1))
